# baseline (speedup 1.0000x reference)
import functools
import math

import jax
import jax.numpy as jnp
from jax import lax
from jax.experimental import pallas as pl
from jax.experimental.pallas import tpu as pltpu
from jax.experimental.pallas import tpu_sc as plsc

D = 128
T = 50
NSEQ = 4096
NROW = NSEQ * T
SCALE = math.sqrt(float(D))

_info = plsc.get_sparse_core_info()
_NC = _info.num_cores
_NS = _info.num_subcores
_NW = _NC * _NS
_L = _info.num_lanes

ROWS_PER_W = NROW // _NW
CH = 128
NCH = ROWS_PER_W // CH
NBUF = 5

_mesh = plsc.VectorSubcoreMesh(core_axis_name="c", subcore_axis_name="s")


@functools.partial(
    pl.kernel,
    mesh=_mesh,
    out_type=jax.ShapeDtypeStruct((NROW, D), jnp.float32),
    scratch_types=(
        [pltpu.VMEM((ROWS_PER_W,), jnp.int32)]
        + [pltpu.VMEM((CH, D), jnp.float32) for _ in range(NBUF)]
        + [pltpu.SemaphoreType.DMA, pltpu.SemaphoreType.DMA]
    ),
    compiler_params=pltpu.CompilerParams(use_tc_tiling_on_sc=True),
)
def _gather_scale(idx_hbm, table_hbm, out_hbm, idx_v, *rest):
    bufs = rest[:NBUF]
    gsem, osem = rest[NBUF], rest[NBUF + 1]
    wid = lax.axis_index("s") * _NC + lax.axis_index("c")
    base = wid * ROWS_PER_W
    pltpu.sync_copy(idx_hbm.at[pl.ds(base, ROWS_PER_W)], idx_v)

    def gather(c, buf):
        return pltpu.make_async_copy(
            table_hbm.at[idx_v.at[pl.ds(c * CH, CH)]], buf, gsem
        )

    def out_copy(c, buf):
        return pltpu.make_async_copy(
            buf, out_hbm.at[pl.ds(base + c * CH, CH)], osem
        )

    def scale(buf):
        def row_body(r, c2):
            for j in range(D // _L):
                sl = pl.ds(j * _L, _L)
                buf[r, sl] = buf[r, sl] * SCALE
            return c2

        lax.fori_loop(0, CH, row_body, 0, unroll=2)

    for k in range(NBUF - 1):
        gather(k, bufs[k]).start()

    def chunk_body(p, carry):
        for b in range(NBUF):
            c = p * NBUF + b
            buf = bufs[b]
            gather(c, buf).wait()

            t = c + NBUF - 1
            tb = bufs[(b + NBUF - 1) % NBUF]

            @pl.when(t < NCH)
            def _():
                @pl.when(t >= NBUF)
                def _():
                    out_copy(t - NBUF, tb).wait()

                gather(t, tb).start()

            scale(buf)
            out_copy(c, buf).start()
        return carry

    lax.fori_loop(0, NCH // NBUF, chunk_body, 0)
    for k in range(NBUF):
        c = NCH - NBUF + k
        out_copy(c, bufs[c % NBUF]).wait()


def kernel(tokens, table):
    idx = tokens.T.reshape(-1).astype(jnp.int32)
    flat = _gather_scale(idx, table)
    return flat.reshape(T, NSEQ, D).transpose(1, 0, 2)
